# Initial kernel scaffold; baseline (speedup 1.0000x reference)
import jax
import jax.numpy as jnp
from jax.experimental import pallas as pl


def kernel(word_embeddings, token_type_ids, type_embeddings, position_embeddings, ln_gamma, ln_beta):
    raise NotImplementedError("write your pallas kernel here")


# speedup vs baseline: 2.2824x; 2.2824x over previous
import functools

import jax
import jax.numpy as jnp
from jax.experimental import pallas as pl
from jax.experimental.pallas import tpu as pltpu

B, S, D = 4, 2048, 1024
EPS = 1e-12
ROWS = 256
assert S % ROWS == 0
SBLK = S // ROWS


def _body(word_ref, tt_ref, type_ref, pos_ref, gamma_ref, beta_ref, out_ref):
    f = tt_ref[0, 0, :].astype(jnp.float32)[:, None]
    t0 = type_ref[0, :]
    tdiff = type_ref[1, :] - t0
    x = word_ref[...] + pos_ref[...] + (t0 + f * tdiff)
    mean = jnp.mean(x, axis=1, keepdims=True)
    xc = x - mean
    var = jnp.mean(xc * xc, axis=1, keepdims=True)
    normed = xc * jax.lax.rsqrt(var + EPS)
    out_ref[...] = normed * gamma_ref[0, :] + beta_ref[0, :]


@jax.jit
def kernel(word_embeddings, token_type_ids, type_embeddings,
           position_embeddings, ln_gamma, ln_beta):
    words = word_embeddings.reshape(B * S, D)
    tt = token_type_ids.reshape(B * S // ROWS, 1, ROWS).astype(jnp.int32)
    gamma = ln_gamma.reshape(1, D)
    beta = ln_beta.reshape(1, D)
    grid = (B * S // ROWS,)
    out = pl.pallas_call(
        _body,
        grid=grid,
        in_specs=[
            pl.BlockSpec((ROWS, D), lambda i: (i, 0)),
            pl.BlockSpec((1, 1, ROWS), lambda i: (i, 0, 0)),
            pl.BlockSpec((2, D), lambda i: (0, 0)),
            pl.BlockSpec((ROWS, D), lambda i: (i % SBLK, 0)),
            pl.BlockSpec((1, D), lambda i: (0, 0)),
            pl.BlockSpec((1, D), lambda i: (0, 0)),
        ],
        out_specs=pl.BlockSpec((ROWS, D), lambda i: (i, 0)),
        out_shape=jax.ShapeDtypeStruct((B * S, D), jnp.float32),
    )(words, tt, type_embeddings, position_embeddings, gamma, beta)
    return out.reshape(B, S, D)
